# Initial kernel scaffold; baseline (speedup 1.0000x reference)
import jax
import jax.numpy as jnp
from jax.experimental import pallas as pl


def kernel(x, edge_index, W1_self, W1_neigh, b1, W2_self, W2_neigh, b2):
    raise NotImplementedError("write your pallas kernel here")


# speedup vs baseline: 3.6116x; 3.6116x over previous
import functools

import jax
import jax.numpy as jnp
from jax import lax
from jax.experimental import pallas as pl
from jax.experimental.pallas import tpu as pltpu
from jax.experimental.pallas import tpu_sc as plsc

N = 10000
E = 160000
D = 256
DH = 128
NSC = 16
N_PAD = 10240
R = N_PAD // NSC
EPT = E // NSC
B = 80
NCHUNK = EPT // B

BM = 400


def _mm2_body(x_ref, ws_ref, wn_ref, b_ref, s_ref, ya_ref, yb_ref):
    xb = x_ref[...]
    s_ref[...] = jnp.dot(xb, ws_ref[...], preferred_element_type=jnp.float32) + b_ref[...]
    y = jnp.dot(xb, wn_ref[...], preferred_element_type=jnp.float32)
    ya_ref[...] = y[:, :DH]
    yb_ref[...] = y[:, DH:]


def _mm2(x, ws, wn, b):
    return pl.pallas_call(
        _mm2_body,
        grid=(N // BM,),
        in_specs=[
            pl.BlockSpec((BM, D), lambda i: (i, 0)),
            pl.BlockSpec((D, D), lambda i: (0, 0)),
            pl.BlockSpec((D, D), lambda i: (0, 0)),
            pl.BlockSpec((1, D), lambda i: (0, 0)),
        ],
        out_specs=[
            pl.BlockSpec((BM, D), lambda i: (i, 0)),
            pl.BlockSpec((BM, DH), lambda i: (i, 0)),
            pl.BlockSpec((BM, DH), lambda i: (i, 0)),
        ],
        out_shape=[
            jax.ShapeDtypeStruct((N_PAD, D), jnp.float32),
            jax.ShapeDtypeStruct((N_PAD, DH), jnp.float32),
            jax.ShapeDtypeStruct((N_PAD, DH), jnp.float32),
        ],
    )(x, ws, wn, b.reshape(1, D))


def _mid_body(s1_ref, aa_ref, ab_ref, cnt_ref, ws_ref, wn_ref, b_ref,
              s2_ref, ya_ref, yb_ref):
    inv = 1.0 / jnp.maximum(cnt_ref[...], 1.0)
    agg = jnp.concatenate([aa_ref[...], ab_ref[...]], axis=1) * inv
    h = jnp.maximum(s1_ref[...] + agg, 0.0)
    s2_ref[...] = jnp.dot(h, ws_ref[...], preferred_element_type=jnp.float32) + b_ref[...]
    y = jnp.dot(h, wn_ref[...], preferred_element_type=jnp.float32)
    ya_ref[...] = y[:, :DH]
    yb_ref[...] = y[:, DH:]


def _mid(s1, aa, ab, cnt2d, ws, wn, b):
    return pl.pallas_call(
        _mid_body,
        grid=(N // BM,),
        in_specs=[
            pl.BlockSpec((BM, D), lambda i: (i, 0)),
            pl.BlockSpec((BM, DH), lambda i: (i, 0)),
            pl.BlockSpec((BM, DH), lambda i: (i, 0)),
            pl.BlockSpec((BM, 1), lambda i: (i, 0)),
            pl.BlockSpec((D, D), lambda i: (0, 0)),
            pl.BlockSpec((D, D), lambda i: (0, 0)),
            pl.BlockSpec((1, D), lambda i: (0, 0)),
        ],
        out_specs=[
            pl.BlockSpec((BM, D), lambda i: (i, 0)),
            pl.BlockSpec((BM, DH), lambda i: (i, 0)),
            pl.BlockSpec((BM, DH), lambda i: (i, 0)),
        ],
        out_shape=[
            jax.ShapeDtypeStruct((N_PAD, D), jnp.float32),
            jax.ShapeDtypeStruct((N_PAD, DH), jnp.float32),
            jax.ShapeDtypeStruct((N_PAD, DH), jnp.float32),
        ],
    )(s1, aa, ab, cnt2d, ws, wn, b.reshape(1, D))


def _fin_body(s2_ref, aa_ref, ab_ref, cnt_ref, o_ref):
    inv = 1.0 / jnp.maximum(cnt_ref[...], 1.0)
    o_ref[...] = s2_ref[...] + jnp.concatenate([aa_ref[...], ab_ref[...]], axis=1) * inv


def _fin(s2, aa, ab, cnt2d):
    return pl.pallas_call(
        _fin_body,
        grid=(N // BM,),
        in_specs=[
            pl.BlockSpec((BM, D), lambda i: (i, 0)),
            pl.BlockSpec((BM, DH), lambda i: (i, 0)),
            pl.BlockSpec((BM, DH), lambda i: (i, 0)),
            pl.BlockSpec((BM, 1), lambda i: (i, 0)),
        ],
        out_specs=pl.BlockSpec((BM, D), lambda i: (i, 0)),
        out_shape=jax.ShapeDtypeStruct((N, D), jnp.float32),
    )(s2, aa, ab, cnt2d)


@functools.lru_cache(maxsize=None)
def _make_segsum(with_cnt):
    mesh = plsc.VectorSubcoreMesh(
        core_axis_name="c", subcore_axis_name="s", num_cores=2, num_subcores=NSC)

    out_type = [
        jax.ShapeDtypeStruct((N_PAD, DH), jnp.float32),
        jax.ShapeDtypeStruct((N_PAD, DH), jnp.float32),
    ]
    scratch = [
        pltpu.VMEM((B,), jnp.int32),
        pltpu.VMEM((B,), jnp.int32),
        pltpu.VMEM((B, DH), jnp.float32),
        pltpu.VMEM_SHARED((N_PAD, DH), jnp.float32),
        pltpu.SemaphoreType.DMA,
    ]
    if with_cnt:
        out_type.append(jax.ShapeDtypeStruct((N_PAD,), jnp.float32))
        scratch += [
            pltpu.VMEM((N_PAD,), jnp.float32),
            pltpu.VMEM((NSC, R), jnp.float32),
            pltpu.VMEM((R,), jnp.float32),
            pltpu.VMEM_SHARED((NSC, N_PAD), jnp.float32),
        ]

    def body(ya, yb, src, dst, zrow, agg_a, agg_b, *rest):
        if with_cnt:
            (cnt_out, sidx, didx, rows, accum, sem,
             cntp, cred, cout, cnt_all) = rest
        else:
            sidx, didx, rows, accum, sem = rest
        c = lax.axis_index("c")
        s = lax.axis_index("s")
        row0 = s * R

        pltpu.sync_copy(zrow.at[pl.ds(row0, R)], accum.at[pl.ds(row0, R)])
        if with_cnt:
            def _zc(i, carry):
                cntp[pl.ds(i * 16, 16)] = jnp.zeros((16,), jnp.float32)
                return carry
            lax.fori_loop(0, N_PAD // 16, _zc, 0)
        plsc.subcore_barrier()

        ones16 = jnp.ones((16,), jnp.float32)
        ebase = s * EPT

        def step(i, carry):
            off = ebase + i * B
            pltpu.sync_copy(src.at[pl.ds(off, B)], sidx)
            pltpu.sync_copy(dst.at[pl.ds(off, B)], didx)

            @pl.when(c == 0)
            def _():
                pltpu.async_copy(ya.at[sidx], rows, sem).wait()

            @pl.when(c == 1)
            def _():
                pltpu.async_copy(yb.at[sidx], rows, sem).wait()

            pltpu.sync_copy(rows, accum.at[didx], add=True)
            if with_cnt:
                @pl.when(c == 0)
                def _():
                    for j in range(B // 16):
                        d16 = didx[pl.ds(j * 16, 16)]
                        plsc.addupdate_scatter(cntp, [d16], ones16)
            return carry

        lax.fori_loop(0, NCHUNK, step, 0)
        plsc.subcore_barrier()

        @pl.when(c == 0)
        def _():
            pltpu.sync_copy(accum.at[pl.ds(row0, R)], agg_a.at[pl.ds(row0, R)])

        @pl.when(c == 1)
        def _():
            pltpu.sync_copy(accum.at[pl.ds(row0, R)], agg_b.at[pl.ds(row0, R)])

        if with_cnt:
            @pl.when(c == 0)
            def _():
                pltpu.sync_copy(cntp, cnt_all.at[s])
            plsc.subcore_barrier()

            @pl.when(c == 0)
            def _():
                pltpu.sync_copy(cnt_all.at[:, pl.ds(row0, R)], cred)

                def red(j, carry):
                    acc = jnp.zeros((16,), jnp.float32)
                    for r in range(NSC):
                        acc = acc + cred[r, pl.ds(j * 16, 16)]
                    cout[pl.ds(j * 16, 16)] = acc
                    return carry
                lax.fori_loop(0, R // 16, red, 0)
                pltpu.sync_copy(cout, cnt_out.at[pl.ds(row0, R)])

    return pl.kernel(
        body, out_type=out_type, mesh=mesh, scratch_types=scratch,
        compiler_params=pltpu.CompilerParams(needs_layout_passes=False))


@jax.jit
def kernel(x, edge_index, W1_self, W1_neigh, b1, W2_self, W2_neigh, b2):
    src = edge_index[0]
    dst = edge_index[1]
    zrow = jnp.zeros((N_PAD, DH), jnp.float32)

    s1, y1a, y1b = _mm2(x, W1_self, W1_neigh, b1)
    agg_a, agg_b, cnt = _make_segsum(True)(y1a, y1b, src, dst, zrow)
    cnt2d = cnt.reshape(N_PAD, 1)
    s2, y2a, y2b = _mid(s1, agg_a, agg_b, cnt2d, W2_self, W2_neigh, b2)
    agg_a2, agg_b2 = _make_segsum(False)(y2a, y2b, src, dst, zrow)
    return _fin(s2, agg_a2, agg_b2, cnt2d)


# speedup vs baseline: 7.8257x; 2.1668x over previous
import functools

import jax
import jax.numpy as jnp
from jax import lax
from jax.experimental import pallas as pl
from jax.experimental.pallas import tpu as pltpu
from jax.experimental.pallas import tpu_sc as plsc

N = 10000
E = 160000
D = 256
DH = 128
NSC = 16
N_PAD = 10240
R = N_PAD // NSC
EPT = E // NSC
B = 80
NCHUNK = EPT // B

BM = 400


def _mm2_body(x_ref, ws_ref, wn_ref, b_ref, s_ref, ya_ref, yb_ref):
    xb = x_ref[...]
    s_ref[...] = jnp.dot(xb, ws_ref[...], preferred_element_type=jnp.float32) + b_ref[...]
    y = jnp.dot(xb, wn_ref[...], preferred_element_type=jnp.float32)
    ya_ref[...] = y[:, :DH]
    yb_ref[...] = y[:, DH:]


def _mm2(x, ws, wn, b):
    return pl.pallas_call(
        _mm2_body,
        grid=(N // BM,),
        in_specs=[
            pl.BlockSpec((BM, D), lambda i: (i, 0)),
            pl.BlockSpec((D, D), lambda i: (0, 0)),
            pl.BlockSpec((D, D), lambda i: (0, 0)),
            pl.BlockSpec((1, D), lambda i: (0, 0)),
        ],
        out_specs=[
            pl.BlockSpec((BM, D), lambda i: (i, 0)),
            pl.BlockSpec((BM, DH), lambda i: (i, 0)),
            pl.BlockSpec((BM, DH), lambda i: (i, 0)),
        ],
        out_shape=[
            jax.ShapeDtypeStruct((N_PAD, D), jnp.float32),
            jax.ShapeDtypeStruct((N_PAD, DH), jnp.float32),
            jax.ShapeDtypeStruct((N_PAD, DH), jnp.float32),
        ],
    )(x, ws, wn, b.reshape(1, D))


def _mid_body(s1_ref, aa_ref, ab_ref, cnt_ref, ws_ref, wn_ref, b_ref,
              s2_ref, ya_ref, yb_ref):
    inv = 1.0 / jnp.maximum(cnt_ref[...], 1.0)
    agg = jnp.concatenate([aa_ref[...], ab_ref[...]], axis=1) * inv
    h = jnp.maximum(s1_ref[...] + agg, 0.0)
    s2_ref[...] = jnp.dot(h, ws_ref[...], preferred_element_type=jnp.float32) + b_ref[...]
    y = jnp.dot(h, wn_ref[...], preferred_element_type=jnp.float32)
    ya_ref[...] = y[:, :DH]
    yb_ref[...] = y[:, DH:]


def _mid(s1, aa, ab, cnt2d, ws, wn, b):
    return pl.pallas_call(
        _mid_body,
        grid=(N // BM,),
        in_specs=[
            pl.BlockSpec((BM, D), lambda i: (i, 0)),
            pl.BlockSpec((BM, DH), lambda i: (i, 0)),
            pl.BlockSpec((BM, DH), lambda i: (i, 0)),
            pl.BlockSpec((BM, 1), lambda i: (i, 0)),
            pl.BlockSpec((D, D), lambda i: (0, 0)),
            pl.BlockSpec((D, D), lambda i: (0, 0)),
            pl.BlockSpec((1, D), lambda i: (0, 0)),
        ],
        out_specs=[
            pl.BlockSpec((BM, D), lambda i: (i, 0)),
            pl.BlockSpec((BM, DH), lambda i: (i, 0)),
            pl.BlockSpec((BM, DH), lambda i: (i, 0)),
        ],
        out_shape=[
            jax.ShapeDtypeStruct((N_PAD, D), jnp.float32),
            jax.ShapeDtypeStruct((N_PAD, DH), jnp.float32),
            jax.ShapeDtypeStruct((N_PAD, DH), jnp.float32),
        ],
    )(s1, aa, ab, cnt2d, ws, wn, b.reshape(1, D))


def _fin_body(s2_ref, aa_ref, ab_ref, cnt_ref, o_ref):
    inv = 1.0 / jnp.maximum(cnt_ref[...], 1.0)
    o_ref[...] = s2_ref[...] + jnp.concatenate([aa_ref[...], ab_ref[...]], axis=1) * inv


def _fin(s2, aa, ab, cnt2d):
    return pl.pallas_call(
        _fin_body,
        grid=(N // BM,),
        in_specs=[
            pl.BlockSpec((BM, D), lambda i: (i, 0)),
            pl.BlockSpec((BM, DH), lambda i: (i, 0)),
            pl.BlockSpec((BM, DH), lambda i: (i, 0)),
            pl.BlockSpec((BM, 1), lambda i: (i, 0)),
        ],
        out_specs=pl.BlockSpec((BM, D), lambda i: (i, 0)),
        out_shape=jax.ShapeDtypeStruct((N, D), jnp.float32),
    )(s2, aa, ab, cnt2d)


ECNT = E // 32


@functools.lru_cache(maxsize=None)
def _make_segsum():
    mesh = plsc.VectorSubcoreMesh(
        core_axis_name="c", subcore_axis_name="s", num_cores=2, num_subcores=NSC)

    out_type = [
        jax.ShapeDtypeStruct((N_PAD, DH), jnp.float32),
        jax.ShapeDtypeStruct((N_PAD, DH), jnp.float32),
    ]
    scratch = [
        pltpu.VMEM((EPT,), jnp.int32),
        pltpu.VMEM((2, B), jnp.int32),
        pltpu.VMEM((2, B, DH), jnp.float32),
        pltpu.VMEM_SHARED((N_PAD, DH), jnp.float32),
        pltpu.SemaphoreType.DMA,
        pltpu.SemaphoreType.DMA,
        pltpu.SemaphoreType.DMA,
    ]

    def body(ya, yb, src2, dst1, zrow, agg_a, agg_b,
             sidx, didx, rows, accum, gsem, ssem, dsem):
        c = lax.axis_index("c")
        s = lax.axis_index("s")
        row0 = s * R
        ebase = s * EPT

        pltpu.sync_copy(zrow.at[pl.ds(row0, R)], accum.at[pl.ds(row0, R)])
        pltpu.sync_copy(src2.at[s], sidx)
        plsc.subcore_barrier()

        def gather(i, buf):
            @pl.when(c == 0)
            def _():
                pltpu.async_copy(ya.at[sidx.at[pl.ds(i * B, B)]], rows.at[buf], gsem)

            @pl.when(c == 1)
            def _():
                pltpu.async_copy(yb.at[sidx.at[pl.ds(i * B, B)]], rows.at[buf], gsem)

        pltpu.async_copy(dst1.at[pl.ds(ebase, B)], didx.at[0], dsem)
        gather(0, 0)

        def step(i, carry):
            cur = lax.rem(i, 2)
            nxt = 1 - cur

            @pl.when(i > 0)
            def _():
                pltpu.make_async_copy(rows.at[nxt], accum.at[didx.at[nxt]], ssem).wait()

            @pl.when(i + 1 < NCHUNK)
            def _():
                pltpu.async_copy(dst1.at[pl.ds(ebase + (i + 1) * B, B)], didx.at[nxt], dsem)
                gather(i + 1, nxt)

            pltpu.make_async_copy(dst1.at[pl.ds(ebase, B)], didx.at[cur], dsem).wait()
            pltpu.make_async_copy(ya.at[sidx.at[pl.ds(0, B)]], rows.at[cur], gsem).wait()
            pltpu.async_copy(rows.at[cur], accum.at[didx.at[cur]], ssem, add=True)
            return carry

        lax.fori_loop(0, NCHUNK, step, 0)
        pltpu.make_async_copy(rows.at[0], accum.at[didx.at[0]], ssem).wait()
        plsc.subcore_barrier()

        @pl.when(c == 0)
        def _():
            pltpu.sync_copy(accum.at[pl.ds(row0, R)], agg_a.at[pl.ds(row0, R)])

        @pl.when(c == 1)
        def _():
            pltpu.sync_copy(accum.at[pl.ds(row0, R)], agg_b.at[pl.ds(row0, R)])

    return pl.kernel(
        body, out_type=out_type, mesh=mesh, scratch_types=scratch,
        compiler_params=pltpu.CompilerParams(needs_layout_passes=False))


@functools.lru_cache(maxsize=None)
def _make_cnt():
    mesh = plsc.VectorSubcoreMesh(
        core_axis_name="c", subcore_axis_name="s", num_cores=2, num_subcores=NSC)

    out_type = [
        jax.ShapeDtypeStruct((N_PAD,), jnp.float32),
        jax.ShapeDtypeStruct((N_PAD,), jnp.float32),
    ]
    scratch = [
        pltpu.VMEM((ECNT,), jnp.int32),
        pltpu.VMEM((N_PAD,), jnp.float32),
        pltpu.VMEM((NSC, R), jnp.float32),
        pltpu.VMEM((R,), jnp.float32),
        pltpu.VMEM_SHARED((NSC, N_PAD), jnp.float32),
    ]

    def body(dst2, c0_out, c1_out, didx, cntp, cred, cout, cnt_all):
        c = lax.axis_index("c")
        s = lax.axis_index("s")
        w = c * NSC + s
        row0 = s * R

        pltpu.sync_copy(dst2.at[w], didx)

        def _zc(i, carry):
            cntp[pl.ds(i * 16, 16)] = jnp.zeros((16,), jnp.float32)
            return carry
        lax.fori_loop(0, N_PAD // 16, _zc, 0)

        ones16 = jnp.ones((16,), jnp.float32)

        def _cc(i, carry):
            d16 = didx[pl.ds(i * 16, 16)]
            plsc.addupdate_scatter(cntp, [d16], ones16)
            return carry
        lax.fori_loop(0, ECNT // 16, _cc, 0)
        d16 = didx[pl.ds(ECNT - 16, 16)]
        lanes = lax.broadcasted_iota(jnp.int32, (16,), 0)
        plsc.addupdate_scatter(cntp, [d16], ones16, mask=lanes >= 8)

        pltpu.sync_copy(cntp, cnt_all.at[s])
        plsc.subcore_barrier()
        pltpu.sync_copy(cnt_all.at[:, pl.ds(row0, R)], cred)

        def red(j, carry):
            acc = jnp.zeros((16,), jnp.float32)
            for r in range(NSC):
                acc = acc + cred[r, pl.ds(j * 16, 16)]
            cout[pl.ds(j * 16, 16)] = acc
            return carry
        lax.fori_loop(0, R // 16, red, 0)

        @pl.when(c == 0)
        def _():
            pltpu.sync_copy(cout, c0_out.at[pl.ds(row0, R)])

        @pl.when(c == 1)
        def _():
            pltpu.sync_copy(cout, c1_out.at[pl.ds(row0, R)])

    return pl.kernel(
        body, out_type=out_type, mesh=mesh, scratch_types=scratch,
        compiler_params=pltpu.CompilerParams(needs_layout_passes=False))


@jax.jit
def kernel(x, edge_index, W1_self, W1_neigh, b1, W2_self, W2_neigh, b2):
    src2 = edge_index[0].reshape(NSC, EPT)
    dst1 = edge_index[1]
    dst2 = edge_index[1].reshape(2 * NSC, ECNT)
    zrow = jnp.zeros((N_PAD, DH), jnp.float32)

    cnt0, cnt1 = _make_cnt()(dst2)
    s1, y1a, y1b = _mm2(x, W1_self, W1_neigh, b1)
    agg_a, agg_b = _make_segsum()(y1a, y1b, src2, dst1, zrow)
    cnt2d = (cnt0 + cnt1).reshape(N_PAD, 1)
    s2, y2a, y2b = _mid(s1, agg_a, agg_b, cnt2d, W2_self, W2_neigh, b2)
    agg_a2, agg_b2 = _make_segsum()(y2a, y2b, src2, dst1, zrow)
    return _fin(s2, agg_a2, agg_b2, cnt2d)
